# baseline (speedup 1.0000x reference)
import functools

import jax
import jax.numpy as jnp
from jax import lax
from jax.experimental import pallas as pl
from jax.experimental.pallas import tpu as pltpu
from jax.experimental.pallas import tpu_sc as plsc

_H = 128
_MAX_NB = 10
_B, _N, _NBOND = 128, 200, 400
_RA = _B * _N
_RB = _B * _NBOND
_NW = 32
_PW = _RA // _NW
_G = 8
_NCH = _PW // _G
_IPC = _G * _MAX_NB
_HC = _H // 16
_NFP = 5

f32 = jnp.float32
i32 = jnp.int32
bf16 = jnp.bfloat16


_BRA = 1024
_BRB = 1024


def _rows(br):
    return pl.BlockSpec((br, _H), lambda i: (i, 0))


def _packed(br):
    return pl.BlockSpec((br, _H // 2), lambda i: (i, 0))


def _full(shape):
    return pl.BlockSpec(shape, lambda i: (0,) * len(shape))


def _dot(x, w):
    return jnp.dot(x.astype(bf16), w.astype(bf16), preferred_element_type=f32)


def _pack2(lo, hi):
    ul = jax.lax.bitcast_convert_type(lo.astype(bf16), jnp.uint16).astype(i32)
    uh = jax.lax.bitcast_convert_type(hi.astype(bf16), jnp.uint16).astype(i32)
    return jax.lax.shift_left(uh, 16) | ul


def _tc_atoms0_body(xa, waf, wu2a_lo, wu2a_hi, bu2_lo, bu2_hi, af_o, t_o):
    af = _dot(xa[...], waf[...])
    af_o[...] = af
    afb = af.astype(bf16)
    t_lo = jnp.dot(afb, wu2a_lo[...].astype(bf16), preferred_element_type=f32)
    t_hi = jnp.dot(afb, wu2a_hi[...].astype(bf16), preferred_element_type=f32)
    t_o[...] = _pack2(t_lo + bu2_lo[...], t_hi + bu2_hi[...])


def _tc_bonds_body(xb, wnb_lo, wnb_hi, wu2b_lo, wu2b_hi, bnb_o, bu2_o):
    x = xb[...].astype(bf16)
    bnb_o[...] = _pack2(jnp.dot(x, wnb_lo[...].astype(bf16), preferred_element_type=f32),
                        jnp.dot(x, wnb_hi[...].astype(bf16), preferred_element_type=f32))
    bu2_o[...] = _pack2(jnp.dot(x, wu2b_lo[...].astype(bf16), preferred_element_type=f32),
                        jnp.dot(x, wu2b_hi[...].astype(bf16), preferred_element_type=f32))


def _tc_update_body(full_tables, af, nei, w1a, w1b, b1,
                    wu2a_lo, wu2a_hi, bu2_lo, bu2_hi, *rest):
    if full_tables:
        wna_lo, wna_hi, wsa, af_o, t_o, na_o, sa_o = rest
    else:
        af_o, t_o = rest
    afn = _dot(af[...], w1a[...]) + _dot(nei[...], w1b[...])
    afn = jnp.maximum(afn + b1[...], 0.0)
    af_o[...] = afn
    afb = afn.astype(bf16)
    t_lo = jnp.dot(afb, wu2a_lo[...].astype(bf16), preferred_element_type=f32)
    t_hi = jnp.dot(afb, wu2a_hi[...].astype(bf16), preferred_element_type=f32)
    t_o[...] = _pack2(t_lo + bu2_lo[...], t_hi + bu2_hi[...])
    if full_tables:
        na_o[...] = _pack2(
            jnp.dot(afb, wna_lo[...].astype(bf16), preferred_element_type=f32),
            jnp.dot(afb, wna_hi[...].astype(bf16), preferred_element_type=f32))
        sa_o[...] = _dot(afn, wsa[...])


def _tc_final_body(af, nei, w1a, w1b, b1, fnei, sa, af_o, k_o):
    afn = _dot(af[...], w1a[...]) + _dot(nei[...], w1b[...])
    af_o[...] = jnp.maximum(afn + b1[...], 0.0)
    k_o[...] = fnei[...] * sa[...]


def _tc_atoms0(xa, waf, wu2a_lo, wu2a_hi, bu2_lo, bu2_hi):
    fa = xa.shape[-1]
    return pl.pallas_call(
        _tc_atoms0_body,
        grid=(_RA // _BRA,),
        in_specs=[pl.BlockSpec((_BRA, fa), lambda i: (i, 0)),
                  _full((fa, _H)), _full((_H, _H // 2)), _full((_H, _H // 2)),
                  _full((1, _H // 2)), _full((1, _H // 2))],
        out_specs=[_rows(_BRA), _packed(_BRA)],
        out_shape=[jax.ShapeDtypeStruct((_RA, _H), f32),
                   jax.ShapeDtypeStruct((_RA, _H // 2), i32)],
    )(xa, waf, wu2a_lo, wu2a_hi, bu2_lo, bu2_hi)


def _tc_bonds(xb, wnb_lo, wnb_hi, wu2b_lo, wu2b_hi):
    fb = xb.shape[-1]
    pds = jax.ShapeDtypeStruct((_RB, _H // 2), i32)
    return pl.pallas_call(
        _tc_bonds_body,
        grid=(_RB // _BRB,),
        in_specs=[pl.BlockSpec((_BRB, fb), lambda i: (i, 0))]
                 + [_full((fb, _H // 2))] * 4,
        out_specs=[_packed(_BRB), _packed(_BRB)],
        out_shape=[pds, pds],
    )(xb, wnb_lo, wnb_hi, wu2b_lo, wu2b_hi)


def _tc_update(full_tables, af, nei, w1a, w1b, b1,
               wu2a_lo, wu2a_hi, bu2_lo, bu2_hi, *weights):
    sds = jax.ShapeDtypeStruct((_RA, _H), f32)
    pds = jax.ShapeDtypeStruct((_RA, _H // 2), i32)
    out_shape = [sds, pds] + ([pds, sds] if full_tables else [])
    out_specs = [_rows(_BRA), _packed(_BRA)] + \
        ([_packed(_BRA), _rows(_BRA)] if full_tables else [])
    wspecs = ([_full((_H, _H // 2))] * 2 + [_full((_H, _H))]) if full_tables else []
    return pl.pallas_call(
        functools.partial(_tc_update_body, full_tables),
        grid=(_RA // _BRA,),
        in_specs=[_rows(_BRA), _rows(_BRA), _full((_H, _H)), _full((_H, _H)),
                  _full((1, _H)), _full((_H, _H // 2)), _full((_H, _H // 2)),
                  _full((1, _H // 2)), _full((1, _H // 2))] + wspecs,
        out_specs=out_specs,
        out_shape=out_shape,
    )(af, nei, w1a, w1b, b1, wu2a_lo, wu2a_hi, bu2_lo, bu2_hi, *weights)


def _tc_final(af, nei, w1a, w1b, b1, fnei, sa):
    sds = jax.ShapeDtypeStruct((_RA, _H), f32)
    return pl.pallas_call(
        _tc_final_body,
        grid=(_RA // _BRA,),
        in_specs=[_rows(_BRA), _rows(_BRA), _full((_H, _H)), _full((_H, _H)),
                  _full((1, _H)), _rows(_BRA), _rows(_BRA)],
        out_specs=[_rows(_BRA), _rows(_BRA)],
        out_shape=[sds, sds],
    )(af, nei, w1a, w1b, b1, fnei, sa)


def _sc_gather_reduce(last_depth):
    info = plsc.get_sparse_core_info()
    nc = info.num_cores

    n_out = 2 if last_depth else 1
    out_type = [jax.ShapeDtypeStruct((_RA, _H), f32)] * n_out
    n_tab = 4 if last_depth else 2
    n_idx = _NCH * _IPC
    npp = n_idx // _NFP
    scratch = (
        [pltpu.VMEM((n_idx,), i32)] * 2
        + [pltpu.VMEM((2 * npp,), i32)]
        + [pltpu.VMEM((_PW + 16,), i32)]
        + [pltpu.VMEM((2, _IPC, _H // 2), i32)] * n_tab
        + [pltpu.VMEM((_G, _H), f32)] * n_out
        + ([pltpu.VMEM((_PW + 16,), f32)] if last_depth else [])
        + [pltpu.SemaphoreType.DMA((2,))] * n_tab
    )

    def body(*refs):
        if last_depth:
            (ta, tb, tna, tnb, ag, bg, nnbh, nmh, nei_o, fnei_o,
             ia, ib, tmp, nnb, ra, rb, rna, rnb, stg, stg2, nm,
             sem_a, sem_b, sem_na, sem_nb) = refs
        else:
            (ta, tb, ag, bg, nnbh, nei_o,
             ia, ib, tmp, nnb, ra, rb, stg, sem_a, sem_b) = refs

        wid = lax.axis_index("s") * nc + lax.axis_index("c")
        lanes = lax.iota(i32, 16)

        def _flatten(dst, src, mul):
            for q in range(_NFP):
                pltpu.sync_copy(src.at[pl.ds(wid * 2 * n_idx + q * 2 * npp, 2 * npp)],
                                tmp)

                def f(t, carry):
                    e0 = 32 * t + 2 * lanes
                    v0 = plsc.load_gather(tmp, [e0])
                    v1 = plsc.load_gather(tmp, [e0 + 1])
                    dst[pl.ds(q * npp + t * 16, 16)] = v0 * mul + v1
                    return carry
                lax.fori_loop(0, npp // 16, f, 0)

        _flatten(ia, ag, _N)
        _flatten(ib, bg, _NBOND)
        pltpu.sync_copy(nnbh.at[pl.ds(wid * _PW, _PW)], nnb.at[pl.ds(0, _PW)])
        if last_depth:
            pltpu.sync_copy(nmh.at[pl.ds(wid * _PW, _PW)], nm.at[pl.ds(0, _PW)])

        def _copies(c, p):
            sa_ = ia.at[pl.ds(c * _IPC, _IPC)]
            sb_ = ib.at[pl.ds(c * _IPC, _IPC)]
            cps = [pltpu.make_async_copy(ta.at[sa_], ra.at[p], sem_a.at[p]),
                   pltpu.make_async_copy(tb.at[sb_], rb.at[p], sem_b.at[p])]
            if last_depth:
                cps += [pltpu.make_async_copy(tna.at[sa_], rna.at[p], sem_na.at[p]),
                        pltpu.make_async_copy(tnb.at[sb_], rnb.at[p], sem_nb.at[p])]
            return cps

        def _start(c, p):
            for cp in _copies(c, p):
                cp.start()

        def _wait(p):
            for cp in _copies(0, p):
                cp.wait()

        zeros = tuple(jnp.zeros((16,), f32) for _ in range(_HC * n_out))
        hw = _HC // 2

        def _lo(v):
            return jax.lax.bitcast_convert_type(jax.lax.shift_left(v, 16), f32)

        def _hi(v):
            return jax.lax.bitcast_convert_type(
                jax.lax.bitwise_and(v, jnp.int32(-65536)), f32)

        def _compute(c, p):
            base = c * _G
            nv = nnb[pl.ds(base, 16)]
            mv = nm[pl.ds(base, 16)] if last_depth else None
            for g in range(_G):
                n_val = nv[g]

                def slot(k, carry):
                    r = g * _MAX_NB + k
                    out = list(carry)
                    for j in range(hw):
                        s = pl.ds(j * 16, 16)
                        va = ra[p, r, s]
                        vb = rb[p, r, s]
                        out[j] = out[j] + jnp.maximum(_lo(va) + _lo(vb), 0.0)
                        out[hw + j] = out[hw + j] + jnp.maximum(_hi(va) + _hi(vb), 0.0)
                    if last_depth:
                        for j in range(hw):
                            s = pl.ds(j * 16, 16)
                            vc = rna[p, r, s]
                            vd = rnb[p, r, s]
                            out[_HC + j] = out[_HC + j] + _lo(vc) * _lo(vd)
                            out[_HC + hw + j] = out[_HC + hw + j] + _hi(vc) * _hi(vd)
                    return tuple(out)

                acc = lax.fori_loop(0, n_val, slot, zeros)
                for j in range(_HC):
                    stg[g, pl.ds(j * 16, 16)] = acc[j]
                if last_depth:
                    m = mv[g]
                    for j in range(_HC):
                        stg2[g, pl.ds(j * 16, 16)] = acc[_HC + j] * m
            row0 = wid * _PW + base
            pltpu.sync_copy(stg, nei_o.at[pl.ds(row0, _G)])
            if last_depth:
                pltpu.sync_copy(stg2, fnei_o.at[pl.ds(row0, _G)])

        _start(0, 0)

        def pair(jj, carry):
            c0 = 2 * jj
            _start(c0 + 1, 1)
            _wait(0)
            _compute(c0, 0)

            @pl.when(c0 + 2 < _NCH)
            def _():
                _start(c0 + 2, 0)

            _wait(1)
            _compute(c0 + 1, 1)
            return carry

        lax.fori_loop(0, _NCH // 2, pair, 0)

    mesh = plsc.VectorSubcoreMesh(core_axis_name="c", subcore_axis_name="s")
    return pl.kernel(body, mesh=mesh, out_type=out_type, scratch_types=scratch,
                     compiler_params=pltpu.CompilerParams(use_tc_tiling_on_sc=False,
                                                          needs_layout_passes=False))


def kernel(input_atom, input_bond, atom_graph, bond_graph, num_nbs, node_mask,
           extra, W_af, W_na, W_nb, W_sa, W_u2, b_u2, W_u1, b_u1):
    fa = input_atom.shape[-1]

    xa = input_atom.reshape(_RA, fa)
    xb = input_bond.reshape(_RB, -1)
    hh = _H // 2
    waf = W_af.astype(f32)
    wu2a = W_u2[:_H].astype(f32)
    wu2b = W_u2[_H:].astype(f32)
    w1a = W_u1[:_H].astype(f32)
    w1b = W_u1[_H:].astype(f32)
    bu2 = b_u2.reshape(1, _H).astype(f32)
    b1 = b_u1.reshape(1, _H).astype(f32)

    ag = atom_graph.astype(i32).reshape(-1)
    bg = bond_graph.astype(i32).reshape(-1)
    nnb = num_nbs.astype(i32).reshape(-1)
    nm = node_mask.astype(f32).reshape(-1)

    sc_mid = _sc_gather_reduce(False)
    sc_last = _sc_gather_reduce(True)

    af, t = _tc_atoms0(xa, waf, wu2a[:, :hh], wu2a[:, hh:],
                       bu2[:, :hh], bu2[:, hh:])
    bnb, bu2t = _tc_bonds(xb, W_nb[:, :hh].astype(f32), W_nb[:, hh:].astype(f32),
                          wu2b[:, :hh], wu2b[:, hh:])

    (nei,) = sc_mid(t, bu2t, ag, bg, nnb)
    af, t = _tc_update(False, af, nei, w1a, w1b, b1,
                       wu2a[:, :hh], wu2a[:, hh:], bu2[:, :hh], bu2[:, hh:])
    (nei,) = sc_mid(t, bu2t, ag, bg, nnb)
    af, t, na, sa = _tc_update(True, af, nei, w1a, w1b, b1,
                               wu2a[:, :hh], wu2a[:, hh:], bu2[:, :hh], bu2[:, hh:],
                               W_na[:, :hh].astype(f32), W_na[:, hh:].astype(f32),
                               W_sa.astype(f32))
    nei, fnei = sc_last(t, bu2t, na, bnb, ag, bg, nnb, nm)
    af, kern = _tc_final(af, nei, w1a, w1b, b1, fnei, sa)

    return (kern.reshape(_B, _N, _H), af.reshape(_B, _N, _H))


# speedup vs baseline: 1.7915x; 1.7915x over previous
import functools

import jax
import jax.numpy as jnp
from jax import lax
from jax.experimental import pallas as pl
from jax.experimental.pallas import tpu as pltpu
from jax.experimental.pallas import tpu_sc as plsc

_H = 128
_MAX_NB = 10
_B, _N, _NBOND = 128, 200, 400
_RA = _B * _N
_RB = _B * _NBOND
_NW = 32
_PW = _RA // _NW
_G = 8
_NCH = _PW // _G
_IPC = _G * _MAX_NB
_HC = _H // 16
_NFP = 5

f32 = jnp.float32
i32 = jnp.int32
bf16 = jnp.bfloat16


_BRA = 3200
_BRB = 3200


def _rows(br):
    return pl.BlockSpec((br, _H), lambda i: (i, 0))


def _packed(br):
    return pl.BlockSpec((br, _H // 2), lambda i: (i, 0))


def _full(shape):
    return pl.BlockSpec(shape, lambda i: (0,) * len(shape))


def _dot(x, w):
    return jnp.dot(x.astype(bf16), w.astype(bf16), preferred_element_type=f32)


def _pack2(lo, hi):
    ul = jax.lax.bitcast_convert_type(lo.astype(bf16), jnp.uint16).astype(i32)
    uh = jax.lax.bitcast_convert_type(hi.astype(bf16), jnp.uint16).astype(i32)
    return jax.lax.shift_left(uh, 16) | ul


def _tc_atoms0_body(xa, waf, wu2a_lo, wu2a_hi, bu2_lo, bu2_hi, af_o, t_o):
    af = _dot(xa[...], waf[...])
    af_o[...] = af
    afb = af.astype(bf16)
    t_lo = jnp.dot(afb, wu2a_lo[...].astype(bf16), preferred_element_type=f32)
    t_hi = jnp.dot(afb, wu2a_hi[...].astype(bf16), preferred_element_type=f32)
    t_o[...] = _pack2(t_lo + bu2_lo[...], t_hi + bu2_hi[...])


def _tc_bonds_body(xb, wnb_lo, wnb_hi, wu2b_lo, wu2b_hi, bnb_o, bu2_o):
    x = xb[...].astype(bf16)
    bnb_o[...] = _pack2(jnp.dot(x, wnb_lo[...].astype(bf16), preferred_element_type=f32),
                        jnp.dot(x, wnb_hi[...].astype(bf16), preferred_element_type=f32))
    bu2_o[...] = _pack2(jnp.dot(x, wu2b_lo[...].astype(bf16), preferred_element_type=f32),
                        jnp.dot(x, wu2b_hi[...].astype(bf16), preferred_element_type=f32))


def _tc_update_body(full_tables, af, nei, w1a, w1b, b1,
                    wu2a_lo, wu2a_hi, bu2_lo, bu2_hi, *rest):
    if full_tables:
        wna_lo, wna_hi, wsa, af_o, t_o, na_o, sa_o = rest
    else:
        af_o, t_o = rest
    afn = _dot(af[...], w1a[...]) + _dot(nei[...], w1b[...])
    afn = jnp.maximum(afn + b1[...], 0.0)
    af_o[...] = afn
    afb = afn.astype(bf16)
    t_lo = jnp.dot(afb, wu2a_lo[...].astype(bf16), preferred_element_type=f32)
    t_hi = jnp.dot(afb, wu2a_hi[...].astype(bf16), preferred_element_type=f32)
    t_o[...] = _pack2(t_lo + bu2_lo[...], t_hi + bu2_hi[...])
    if full_tables:
        na_o[...] = _pack2(
            jnp.dot(afb, wna_lo[...].astype(bf16), preferred_element_type=f32),
            jnp.dot(afb, wna_hi[...].astype(bf16), preferred_element_type=f32))
        sa_o[...] = _dot(afn, wsa[...])


def _tc_final_body(af, nei, w1a, w1b, b1, fnei, sa, af_o, k_o):
    afn = _dot(af[...], w1a[...]) + _dot(nei[...], w1b[...])
    af_o[...] = jnp.maximum(afn + b1[...], 0.0)
    k_o[...] = fnei[...] * sa[...]


def _tc_atoms0(xa, waf, wu2a_lo, wu2a_hi, bu2_lo, bu2_hi):
    fa = xa.shape[-1]
    return pl.pallas_call(
        _tc_atoms0_body,
        grid=(_RA // _BRA,),
        in_specs=[pl.BlockSpec((_BRA, fa), lambda i: (i, 0)),
                  _full((fa, _H)), _full((_H, _H // 2)), _full((_H, _H // 2)),
                  _full((1, _H // 2)), _full((1, _H // 2))],
        out_specs=[_rows(_BRA), _packed(_BRA)],
        out_shape=[jax.ShapeDtypeStruct((_RA, _H), f32),
                   jax.ShapeDtypeStruct((_RA, _H // 2), i32)],
    )(xa, waf, wu2a_lo, wu2a_hi, bu2_lo, bu2_hi)


def _tc_bonds(xb, wnb_lo, wnb_hi, wu2b_lo, wu2b_hi):
    fb = xb.shape[-1]
    pds = jax.ShapeDtypeStruct((_RB, _H // 2), i32)
    return pl.pallas_call(
        _tc_bonds_body,
        grid=(_RB // _BRB,),
        in_specs=[pl.BlockSpec((_BRB, fb), lambda i: (i, 0))]
                 + [_full((fb, _H // 2))] * 4,
        out_specs=[_packed(_BRB), _packed(_BRB)],
        out_shape=[pds, pds],
    )(xb, wnb_lo, wnb_hi, wu2b_lo, wu2b_hi)


def _tc_update(full_tables, af, nei, w1a, w1b, b1,
               wu2a_lo, wu2a_hi, bu2_lo, bu2_hi, *weights):
    sds = jax.ShapeDtypeStruct((_RA, _H), f32)
    pds = jax.ShapeDtypeStruct((_RA, _H // 2), i32)
    out_shape = [sds, pds] + ([pds, sds] if full_tables else [])
    out_specs = [_rows(_BRA), _packed(_BRA)] + \
        ([_packed(_BRA), _rows(_BRA)] if full_tables else [])
    wspecs = ([_full((_H, _H // 2))] * 2 + [_full((_H, _H))]) if full_tables else []
    return pl.pallas_call(
        functools.partial(_tc_update_body, full_tables),
        grid=(_RA // _BRA,),
        in_specs=[_rows(_BRA), _rows(_BRA), _full((_H, _H)), _full((_H, _H)),
                  _full((1, _H)), _full((_H, _H // 2)), _full((_H, _H // 2)),
                  _full((1, _H // 2)), _full((1, _H // 2))] + wspecs,
        out_specs=out_specs,
        out_shape=out_shape,
    )(af, nei, w1a, w1b, b1, wu2a_lo, wu2a_hi, bu2_lo, bu2_hi, *weights)


def _tc_final(af, nei, w1a, w1b, b1, fnei, sa):
    sds = jax.ShapeDtypeStruct((_RA, _H), f32)
    return pl.pallas_call(
        _tc_final_body,
        grid=(_RA // _BRA,),
        in_specs=[_rows(_BRA), _rows(_BRA), _full((_H, _H)), _full((_H, _H)),
                  _full((1, _H)), _rows(_BRA), _rows(_BRA)],
        out_specs=[_rows(_BRA), _rows(_BRA)],
        out_shape=[sds, sds],
    )(af, nei, w1a, w1b, b1, fnei, sa)


def _sc_gather_reduce(last_depth):
    info = plsc.get_sparse_core_info()
    nc = info.num_cores

    n_out = 2 if last_depth else 1
    out_type = [jax.ShapeDtypeStruct((_RA, _H), f32)] * n_out
    n_tab = 4 if last_depth else 2
    n_idx = _NCH * _IPC
    scratch = (
        [pltpu.VMEM((n_idx,), i32)] * 2
        + [pltpu.VMEM((_PW + 16,), i32)]
        + [pltpu.VMEM((2, _IPC, _H // 2), i32)] * n_tab
        + [pltpu.VMEM((_G, _H), f32)] * n_out
        + ([pltpu.VMEM((_PW + 16,), f32)] if last_depth else [])
        + [pltpu.SemaphoreType.DMA((2,))] * n_tab
    )

    def body(*refs):
        if last_depth:
            (ta, tb, tna, tnb, ag, bg, nnbh, nmh, nei_o, fnei_o,
             ia, ib, nnb, ra, rb, rna, rnb, stg, stg2, nm,
             sem_a, sem_b, sem_na, sem_nb) = refs
        else:
            (ta, tb, ag, bg, nnbh, nei_o,
             ia, ib, nnb, ra, rb, stg, sem_a, sem_b) = refs

        wid = lax.axis_index("s") * nc + lax.axis_index("c")

        pltpu.sync_copy(ag.at[pl.ds(wid * n_idx, n_idx)], ia)
        pltpu.sync_copy(bg.at[pl.ds(wid * n_idx, n_idx)], ib)
        pltpu.sync_copy(nnbh.at[pl.ds(wid * _PW, _PW)], nnb.at[pl.ds(0, _PW)])
        if last_depth:
            pltpu.sync_copy(nmh.at[pl.ds(wid * _PW, _PW)], nm.at[pl.ds(0, _PW)])

        def _copies(c, p):
            sa_ = ia.at[pl.ds(c * _IPC, _IPC)]
            sb_ = ib.at[pl.ds(c * _IPC, _IPC)]
            cps = [pltpu.make_async_copy(ta.at[sa_], ra.at[p], sem_a.at[p]),
                   pltpu.make_async_copy(tb.at[sb_], rb.at[p], sem_b.at[p])]
            if last_depth:
                cps += [pltpu.make_async_copy(tna.at[sa_], rna.at[p], sem_na.at[p]),
                        pltpu.make_async_copy(tnb.at[sb_], rnb.at[p], sem_nb.at[p])]
            return cps

        def _start(c, p):
            for cp in _copies(c, p):
                cp.start()

        def _wait(p):
            for cp in _copies(0, p):
                cp.wait()

        zeros = tuple(jnp.zeros((16,), f32) for _ in range(_HC * n_out))
        hw = _HC // 2

        def _lo(v):
            return jax.lax.bitcast_convert_type(jax.lax.shift_left(v, 16), f32)

        def _hi(v):
            return jax.lax.bitcast_convert_type(
                jax.lax.bitwise_and(v, jnp.int32(-65536)), f32)

        def _compute(c, p):
            base = c * _G
            nv = nnb[pl.ds(base, 16)]
            mv = nm[pl.ds(base, 16)] if last_depth else None
            for g in range(_G):
                n_val = nv[g]

                def slot(k, carry):
                    r = g * _MAX_NB + k
                    out = list(carry)
                    for j in range(hw):
                        s = pl.ds(j * 16, 16)
                        va = ra[p, r, s]
                        vb = rb[p, r, s]
                        out[j] = out[j] + jnp.maximum(_lo(va) + _lo(vb), 0.0)
                        out[hw + j] = out[hw + j] + jnp.maximum(_hi(va) + _hi(vb), 0.0)
                    if last_depth:
                        for j in range(hw):
                            s = pl.ds(j * 16, 16)
                            vc = rna[p, r, s]
                            vd = rnb[p, r, s]
                            out[_HC + j] = out[_HC + j] + _lo(vc) * _lo(vd)
                            out[_HC + hw + j] = out[_HC + hw + j] + _hi(vc) * _hi(vd)
                    return tuple(out)

                acc = lax.fori_loop(0, n_val, slot, zeros)
                for j in range(_HC):
                    stg[g, pl.ds(j * 16, 16)] = acc[j]
                if last_depth:
                    m = mv[g]
                    for j in range(_HC):
                        stg2[g, pl.ds(j * 16, 16)] = acc[_HC + j] * m
            row0 = wid * _PW + base
            pltpu.sync_copy(stg, nei_o.at[pl.ds(row0, _G)])
            if last_depth:
                pltpu.sync_copy(stg2, fnei_o.at[pl.ds(row0, _G)])

        _start(0, 0)

        def pair(jj, carry):
            c0 = 2 * jj
            _start(c0 + 1, 1)
            _wait(0)
            _compute(c0, 0)

            @pl.when(c0 + 2 < _NCH)
            def _():
                _start(c0 + 2, 0)

            _wait(1)
            _compute(c0 + 1, 1)
            return carry

        lax.fori_loop(0, _NCH // 2, pair, 0)

    mesh = plsc.VectorSubcoreMesh(core_axis_name="c", subcore_axis_name="s")
    return pl.kernel(body, mesh=mesh, out_type=out_type, scratch_types=scratch,
                     compiler_params=pltpu.CompilerParams(use_tc_tiling_on_sc=False,
                                                          needs_layout_passes=False))


def kernel(input_atom, input_bond, atom_graph, bond_graph, num_nbs, node_mask,
           extra, W_af, W_na, W_nb, W_sa, W_u2, b_u2, W_u1, b_u1):
    fa = input_atom.shape[-1]

    xa = input_atom.reshape(_RA, fa)
    xb = input_bond.reshape(_RB, -1)
    hh = _H // 2
    waf = W_af.astype(f32)
    wu2a = W_u2[:_H].astype(f32)
    wu2b = W_u2[_H:].astype(f32)
    w1a = W_u1[:_H].astype(f32)
    w1b = W_u1[_H:].astype(f32)
    bu2 = b_u2.reshape(1, _H).astype(f32)
    b1 = b_u1.reshape(1, _H).astype(f32)

    ag = (atom_graph[..., 0].astype(i32) * _N
          + atom_graph[..., 1].astype(i32)).reshape(-1)
    bg = (bond_graph[..., 0].astype(i32) * _NBOND
          + bond_graph[..., 1].astype(i32)).reshape(-1)
    nnb = num_nbs.astype(i32).reshape(-1)
    nm = node_mask.astype(f32).reshape(-1)

    sc_mid = _sc_gather_reduce(False)
    sc_last = _sc_gather_reduce(True)

    af, t = _tc_atoms0(xa, waf, wu2a[:, :hh], wu2a[:, hh:],
                       bu2[:, :hh], bu2[:, hh:])
    bnb, bu2t = _tc_bonds(xb, W_nb[:, :hh].astype(f32), W_nb[:, hh:].astype(f32),
                          wu2b[:, :hh], wu2b[:, hh:])

    (nei,) = sc_mid(t, bu2t, ag, bg, nnb)
    af, t = _tc_update(False, af, nei, w1a, w1b, b1,
                       wu2a[:, :hh], wu2a[:, hh:], bu2[:, :hh], bu2[:, hh:])
    (nei,) = sc_mid(t, bu2t, ag, bg, nnb)
    af, t, na, sa = _tc_update(True, af, nei, w1a, w1b, b1,
                               wu2a[:, :hh], wu2a[:, hh:], bu2[:, :hh], bu2[:, hh:],
                               W_na[:, :hh].astype(f32), W_na[:, hh:].astype(f32),
                               W_sa.astype(f32))
    nei, fnei = sc_last(t, bu2t, na, bnb, ag, bg, nnb, nm)
    af, kern = _tc_final(af, nei, w1a, w1b, b1, fnei, sa)

    return (kern.reshape(_B, _N, _H), af.reshape(_B, _N, _H))


# speedup vs baseline: 1.8128x; 1.0119x over previous
import functools

import jax
import jax.numpy as jnp
from jax import lax
from jax.experimental import pallas as pl
from jax.experimental.pallas import tpu as pltpu
from jax.experimental.pallas import tpu_sc as plsc

_H = 128
_MAX_NB = 10
_B, _N, _NBOND = 128, 200, 400
_RA = _B * _N
_RB = _B * _NBOND
_NW = 32
_PW = _RA // _NW
_G = 8
_NCH = _PW // _G
_IPC = _G * _MAX_NB
_HC = _H // 16
_NFP = 5

f32 = jnp.float32
i32 = jnp.int32
bf16 = jnp.bfloat16


_BRA = 3200
_BRB = 3200


def _rows(br):
    return pl.BlockSpec((br, _H), lambda i: (i, 0))


def _packed(br):
    return pl.BlockSpec((br, _H // 2), lambda i: (i, 0))


def _full(shape):
    return pl.BlockSpec(shape, lambda i: (0,) * len(shape))


def _dot(x, w):
    return jnp.dot(x.astype(bf16), w.astype(bf16), preferred_element_type=f32)


def _pack2(lo, hi):
    ul = jax.lax.bitcast_convert_type(lo.astype(bf16), jnp.uint16).astype(i32)
    uh = jax.lax.bitcast_convert_type(hi.astype(bf16), jnp.uint16).astype(i32)
    return jax.lax.shift_left(uh, 16) | ul


def _tc_atoms0_body(xa, waf, wu2a_lo, wu2a_hi, bu2_lo, bu2_hi, af_o, t_o):
    af = _dot(xa[...], waf[...])
    afb = af.astype(bf16)
    af_o[...] = afb
    t_lo = jnp.dot(afb, wu2a_lo[...].astype(bf16), preferred_element_type=f32)
    t_hi = jnp.dot(afb, wu2a_hi[...].astype(bf16), preferred_element_type=f32)
    t_o[...] = _pack2(t_lo + bu2_lo[...], t_hi + bu2_hi[...])


def _tc_bonds_body(xb, wnb_lo, wnb_hi, wu2b_lo, wu2b_hi, bnb_o, bu2_o):
    x = xb[...].astype(bf16)
    bnb_o[...] = _pack2(jnp.dot(x, wnb_lo[...].astype(bf16), preferred_element_type=f32),
                        jnp.dot(x, wnb_hi[...].astype(bf16), preferred_element_type=f32))
    bu2_o[...] = _pack2(jnp.dot(x, wu2b_lo[...].astype(bf16), preferred_element_type=f32),
                        jnp.dot(x, wu2b_hi[...].astype(bf16), preferred_element_type=f32))


def _tc_update_body(full_tables, af, nei, w1a, w1b, b1,
                    wu2a_lo, wu2a_hi, bu2_lo, bu2_hi, *rest):
    if full_tables:
        wna_lo, wna_hi, wsa, af_o, t_o, na_o, sa_o = rest
    else:
        af_o, t_o = rest
    afn = _dot(af[...], w1a[...]) + _dot(nei[...], w1b[...])
    afn = jnp.maximum(afn + b1[...], 0.0)
    afb = afn.astype(bf16)
    af_o[...] = afb
    t_lo = jnp.dot(afb, wu2a_lo[...].astype(bf16), preferred_element_type=f32)
    t_hi = jnp.dot(afb, wu2a_hi[...].astype(bf16), preferred_element_type=f32)
    t_o[...] = _pack2(t_lo + bu2_lo[...], t_hi + bu2_hi[...])
    if full_tables:
        na_o[...] = _pack2(
            jnp.dot(afb, wna_lo[...].astype(bf16), preferred_element_type=f32),
            jnp.dot(afb, wna_hi[...].astype(bf16), preferred_element_type=f32))
        sa_o[...] = _dot(afn, wsa[...])


def _tc_final_body(af, nei, w1a, w1b, b1, fnei, sa, af_o, k_o):
    afn = _dot(af[...], w1a[...]) + _dot(nei[...], w1b[...])
    af_o[...] = jnp.maximum(afn + b1[...], 0.0)
    k_o[...] = fnei[...] * sa[...]


def _tc_atoms0(xa, waf, wu2a_lo, wu2a_hi, bu2_lo, bu2_hi):
    fa = xa.shape[-1]
    return pl.pallas_call(
        _tc_atoms0_body,
        grid=(_RA // _BRA,),
        in_specs=[pl.BlockSpec((_BRA, fa), lambda i: (i, 0)),
                  _full((fa, _H)), _full((_H, _H // 2)), _full((_H, _H // 2)),
                  _full((1, _H // 2)), _full((1, _H // 2))],
        out_specs=[_rows(_BRA), _packed(_BRA)],
        out_shape=[jax.ShapeDtypeStruct((_RA, _H), bf16),
                   jax.ShapeDtypeStruct((_RA, _H // 2), i32)],
    )(xa, waf, wu2a_lo, wu2a_hi, bu2_lo, bu2_hi)


def _tc_bonds(xb, wnb_lo, wnb_hi, wu2b_lo, wu2b_hi):
    fb = xb.shape[-1]
    pds = jax.ShapeDtypeStruct((_RB, _H // 2), i32)
    return pl.pallas_call(
        _tc_bonds_body,
        grid=(_RB // _BRB,),
        in_specs=[pl.BlockSpec((_BRB, fb), lambda i: (i, 0))]
                 + [_full((fb, _H // 2))] * 4,
        out_specs=[_packed(_BRB), _packed(_BRB)],
        out_shape=[pds, pds],
    )(xb, wnb_lo, wnb_hi, wu2b_lo, wu2b_hi)


def _tc_update(full_tables, af, nei, w1a, w1b, b1,
               wu2a_lo, wu2a_hi, bu2_lo, bu2_hi, *weights):
    bds = jax.ShapeDtypeStruct((_RA, _H), bf16)
    sds = jax.ShapeDtypeStruct((_RA, _H), f32)
    pds = jax.ShapeDtypeStruct((_RA, _H // 2), i32)
    out_shape = [bds, pds] + ([pds, sds] if full_tables else [])
    out_specs = [_rows(_BRA), _packed(_BRA)] + \
        ([_packed(_BRA), _rows(_BRA)] if full_tables else [])
    wspecs = ([_full((_H, _H // 2))] * 2 + [_full((_H, _H))]) if full_tables else []
    return pl.pallas_call(
        functools.partial(_tc_update_body, full_tables),
        grid=(_RA // _BRA,),
        in_specs=[_rows(_BRA), _rows(_BRA), _full((_H, _H)), _full((_H, _H)),
                  _full((1, _H)), _full((_H, _H // 2)), _full((_H, _H // 2)),
                  _full((1, _H // 2)), _full((1, _H // 2))] + wspecs,
        out_specs=out_specs,
        out_shape=out_shape,
    )(af, nei, w1a, w1b, b1, wu2a_lo, wu2a_hi, bu2_lo, bu2_hi, *weights)


def _tc_final(af, nei, w1a, w1b, b1, fnei, sa):
    sds = jax.ShapeDtypeStruct((_RA, _H), f32)
    return pl.pallas_call(
        _tc_final_body,
        grid=(_RA // _BRA,),
        in_specs=[_rows(_BRA), _rows(_BRA), _full((_H, _H)), _full((_H, _H)),
                  _full((1, _H)), _rows(_BRA), _rows(_BRA)],
        out_specs=[_rows(_BRA), _rows(_BRA)],
        out_shape=[sds, sds],
    )(af, nei, w1a, w1b, b1, fnei, sa)


def _sc_gather_reduce(last_depth):
    info = plsc.get_sparse_core_info()
    nc = info.num_cores

    n_out = 2 if last_depth else 1
    out_type = [jax.ShapeDtypeStruct((_RA, _H), f32)] * n_out
    n_tab = 4 if last_depth else 2
    n_idx = _NCH * _IPC
    scratch = (
        [pltpu.VMEM((n_idx,), i32)] * 2
        + [pltpu.VMEM((_PW + 16,), i32)]
        + [pltpu.VMEM((2, _IPC, _H // 2), i32)] * n_tab
        + [pltpu.VMEM((_G, _H), f32)] * n_out
        + ([pltpu.VMEM((_PW + 16,), f32)] if last_depth else [])
        + [pltpu.SemaphoreType.DMA((2,))] * n_tab
    )

    def body(*refs):
        if last_depth:
            (ta, tb, tna, tnb, ag, bg, nnbh, nmh, nei_o, fnei_o,
             ia, ib, nnb, ra, rb, rna, rnb, stg, stg2, nm,
             sem_a, sem_b, sem_na, sem_nb) = refs
        else:
            (ta, tb, ag, bg, nnbh, nei_o,
             ia, ib, nnb, ra, rb, stg, sem_a, sem_b) = refs

        wid = lax.axis_index("s") * nc + lax.axis_index("c")

        pltpu.sync_copy(ag.at[pl.ds(wid * n_idx, n_idx)], ia)
        pltpu.sync_copy(bg.at[pl.ds(wid * n_idx, n_idx)], ib)
        pltpu.sync_copy(nnbh.at[pl.ds(wid * _PW, _PW)], nnb.at[pl.ds(0, _PW)])
        if last_depth:
            pltpu.sync_copy(nmh.at[pl.ds(wid * _PW, _PW)], nm.at[pl.ds(0, _PW)])

        def _copies(c, p):
            sa_ = ia.at[pl.ds(c * _IPC, _IPC)]
            sb_ = ib.at[pl.ds(c * _IPC, _IPC)]
            cps = [pltpu.make_async_copy(ta.at[sa_], ra.at[p], sem_a.at[p]),
                   pltpu.make_async_copy(tb.at[sb_], rb.at[p], sem_b.at[p])]
            if last_depth:
                cps += [pltpu.make_async_copy(tna.at[sa_], rna.at[p], sem_na.at[p]),
                        pltpu.make_async_copy(tnb.at[sb_], rnb.at[p], sem_nb.at[p])]
            return cps

        def _start(c, p):
            for cp in _copies(c, p):
                cp.start()

        def _wait(p):
            for cp in _copies(0, p):
                cp.wait()

        zeros = tuple(jnp.zeros((16,), f32) for _ in range(_HC * n_out))
        hw = _HC // 2

        def _lo(v):
            return jax.lax.bitcast_convert_type(jax.lax.shift_left(v, 16), f32)

        def _hi(v):
            return jax.lax.bitcast_convert_type(
                jax.lax.bitwise_and(v, jnp.int32(-65536)), f32)

        def _compute(c, p):
            base = c * _G
            nv = nnb[pl.ds(base, 16)]
            mv = nm[pl.ds(base, 16)] if last_depth else None
            for g in range(_G):
                n_val = nv[g]

                def slot(k, carry):
                    r = g * _MAX_NB + k
                    out = list(carry)
                    for j in range(hw):
                        s = pl.ds(j * 16, 16)
                        va = ra[p, r, s]
                        vb = rb[p, r, s]
                        out[j] = out[j] + jnp.maximum(_lo(va) + _lo(vb), 0.0)
                        out[hw + j] = out[hw + j] + jnp.maximum(_hi(va) + _hi(vb), 0.0)
                    if last_depth:
                        for j in range(hw):
                            s = pl.ds(j * 16, 16)
                            vc = rna[p, r, s]
                            vd = rnb[p, r, s]
                            out[_HC + j] = out[_HC + j] + _lo(vc) * _lo(vd)
                            out[_HC + hw + j] = out[_HC + hw + j] + _hi(vc) * _hi(vd)
                    return tuple(out)

                acc = lax.fori_loop(0, n_val, slot, zeros)
                for j in range(_HC):
                    stg[g, pl.ds(j * 16, 16)] = acc[j]
                if last_depth:
                    m = mv[g]
                    for j in range(_HC):
                        stg2[g, pl.ds(j * 16, 16)] = acc[_HC + j] * m
            row0 = wid * _PW + base
            pltpu.sync_copy(stg, nei_o.at[pl.ds(row0, _G)])
            if last_depth:
                pltpu.sync_copy(stg2, fnei_o.at[pl.ds(row0, _G)])

        _start(0, 0)

        def pair(jj, carry):
            c0 = 2 * jj
            _start(c0 + 1, 1)
            _wait(0)
            _compute(c0, 0)

            @pl.when(c0 + 2 < _NCH)
            def _():
                _start(c0 + 2, 0)

            _wait(1)
            _compute(c0 + 1, 1)
            return carry

        lax.fori_loop(0, _NCH // 2, pair, 0)

    mesh = plsc.VectorSubcoreMesh(core_axis_name="c", subcore_axis_name="s")
    return pl.kernel(body, mesh=mesh, out_type=out_type, scratch_types=scratch,
                     compiler_params=pltpu.CompilerParams(use_tc_tiling_on_sc=False,
                                                          needs_layout_passes=False))


def kernel(input_atom, input_bond, atom_graph, bond_graph, num_nbs, node_mask,
           extra, W_af, W_na, W_nb, W_sa, W_u2, b_u2, W_u1, b_u1):
    fa = input_atom.shape[-1]

    xa = input_atom.reshape(_RA, fa)
    xb = input_bond.reshape(_RB, -1)
    hh = _H // 2
    waf = W_af.astype(f32)
    wu2a = W_u2[:_H].astype(f32)
    wu2b = W_u2[_H:].astype(f32)
    w1a = W_u1[:_H].astype(f32)
    w1b = W_u1[_H:].astype(f32)
    bu2 = b_u2.reshape(1, _H).astype(f32)
    b1 = b_u1.reshape(1, _H).astype(f32)

    ag = (atom_graph[..., 0].astype(i32) * _N
          + atom_graph[..., 1].astype(i32)).reshape(-1)
    bg = (bond_graph[..., 0].astype(i32) * _NBOND
          + bond_graph[..., 1].astype(i32)).reshape(-1)
    nnb = num_nbs.astype(i32).reshape(-1)
    nm = node_mask.astype(f32).reshape(-1)

    sc_mid = _sc_gather_reduce(False)
    sc_last = _sc_gather_reduce(True)

    af, t = _tc_atoms0(xa, waf, wu2a[:, :hh], wu2a[:, hh:],
                       bu2[:, :hh], bu2[:, hh:])
    bnb, bu2t = _tc_bonds(xb, W_nb[:, :hh].astype(f32), W_nb[:, hh:].astype(f32),
                          wu2b[:, :hh], wu2b[:, hh:])

    (nei,) = sc_mid(t, bu2t, ag, bg, nnb)
    af, t = _tc_update(False, af, nei, w1a, w1b, b1,
                       wu2a[:, :hh], wu2a[:, hh:], bu2[:, :hh], bu2[:, hh:])
    (nei,) = sc_mid(t, bu2t, ag, bg, nnb)
    af, t, na, sa = _tc_update(True, af, nei, w1a, w1b, b1,
                               wu2a[:, :hh], wu2a[:, hh:], bu2[:, :hh], bu2[:, hh:],
                               W_na[:, :hh].astype(f32), W_na[:, hh:].astype(f32),
                               W_sa.astype(f32))
    nei, fnei = sc_last(t, bu2t, na, bnb, ag, bg, nnb, nm)
    af, kern = _tc_final(af, nei, w1a, w1b, b1, fnei, sa)

    return (kern.reshape(_B, _N, _H), af.reshape(_B, _N, _H))
